# baseline (device time: 270346 ns/iter reference)
import jax
import jax.numpy as jnp
from jax import lax
from jax.experimental import pallas as pl
from jax.experimental.pallas import tpu as pltpu

W = 32
BLK = 8
PIECES = 8


def _body(x_ref, dAT_ref, B_ref, L_ref, out_ref,
          hx_ref, hB_ref, h_ref, stg_out_ref, stg_in_ref,
          halo_send_sems, halo_recv_sems, pc_send_sems, pc_recv_sems,
          ack_sem, exit_sem):
    Bb, S, D = x_ref.shape
    N = B_ref.shape[-1]
    Sh = S // 2
    PC = Sh // PIECES
    my_x = lax.axis_index("x")
    my_y = lax.axis_index("y")
    t0 = my_y * Sh

    barrier = pltpu.get_barrier_semaphore()
    pl.semaphore_signal(barrier, inc=1, device_id=(1 - my_x, my_y),
                        device_id_type=pl.DeviceIdType.MESH)
    pl.semaphore_signal(barrier, inc=1, device_id=(my_x, 1 - my_y),
                        device_id_type=pl.DeviceIdType.MESH)
    pl.semaphore_wait(barrier, 2)

    def halo_rdmas():
        rx = pltpu.make_async_remote_copy(
            src_ref=x_ref.at[:, pl.ds(S - W, W), :],
            dst_ref=hx_ref,
            send_sem=halo_send_sems.at[0],
            recv_sem=halo_recv_sems.at[0],
            device_id=(1, my_y),
            device_id_type=pl.DeviceIdType.MESH,
        )
        rb = pltpu.make_async_remote_copy(
            src_ref=B_ref.at[:, pl.ds(S - W, W), :],
            dst_ref=hB_ref,
            send_sem=halo_send_sems.at[1],
            recv_sem=halo_recv_sems.at[1],
            device_id=(1, my_y),
            device_id_type=pl.DeviceIdType.MESH,
        )
        return rx, rb

    DC = D // PIECES

    def piece_rdma(p):
        return pltpu.make_async_remote_copy(
            src_ref=stg_out_ref.at[:, :, pl.ds(p * DC, DC)],
            dst_ref=stg_in_ref.at[:, :, pl.ds(p * DC, DC)],
            send_sem=pc_send_sems.at[p],
            recv_sem=pc_recv_sems.at[p],
            device_id=(my_x, 1 - my_y),
            device_id_type=pl.DeviceIdType.MESH,
        )

    @pl.when(my_x == 0)
    def _():
        rx, rb = halo_rdmas()
        rx.start()
        rb.start()

    h_ref[...] = jnp.zeros_like(h_ref)
    dAT = dAT_ref[...][None]

    @pl.when(my_x == 1)
    def _():
        rx, rb = halo_rdmas()
        rx.wait_recv()
        rb.wait_recv()
        pl.semaphore_signal(ack_sem, inc=1, device_id=(0, my_y),
                            device_id_type=pl.DeviceIdType.MESH)

    @pl.when(my_y == 1)
    def _():
        hx_ref[...] = x_ref[:, Sh - W:Sh, :]
        hB_ref[...] = B_ref[:, Sh - W:Sh, :]

    @pl.when(jnp.logical_or(my_x == 1, my_y == 1))
    def _():
        hbt = jnp.transpose(hB_ref[...], (0, 2, 1))
        for i in range(W):
            xt = hx_ref[:, i:i + 1, :]
            bt = hbt[:, :, i:i + 1]
            h_ref[...] = h_ref[...] * dAT + xt * bt

    def step(c, h, p):
        d0 = p * DC
        base = t0 + c * BLK
        xblk = x_ref[:, pl.ds(base, BLK), d0:d0 + DC]
        bbt = jnp.transpose(B_ref[:, pl.ds(base, BLK), :], (0, 2, 1))
        lblk = L_ref[pl.ds(base, BLK)]
        dAc = dAT[:, :, d0:d0 + DC]
        ys = []
        for k in range(BLK):
            xt = xblk[:, k:k + 1, :]
            bt = bbt[:, :, k:k + 1]
            h = h * dAc + xt * bt
            y = jnp.dot(lblk[k], h.reshape(Bb * N, DC),
                        preferred_element_type=jnp.float32)
            ys.append(y[:, None, :])
        yblk = jnp.concatenate(ys, axis=1)
        out_ref[:, pl.ds(base, BLK), d0:d0 + DC] = yblk
        stg_out_ref[:, pl.ds(c * BLK, BLK), d0:d0 + DC] = yblk.astype(jnp.bfloat16)
        return h

    for p in range(PIECES):
        lax.fori_loop(0, Sh // BLK, lambda c, h, p=p: step(c, h, p),
                      h_ref[:, :, p * DC:(p + 1) * DC])
        piece_rdma(p).start()

    for p in range(PIECES):
        piece_rdma(p).wait_recv()
        out_ref[:, pl.ds((1 - my_y) * Sh, Sh), p * DC:(p + 1) * DC] = (
            stg_in_ref[:, :, p * DC:(p + 1) * DC].astype(jnp.float32))
    for p in range(PIECES):
        piece_rdma(p).wait_send()

    @pl.when(my_x == 0)
    def _():
        rx, rb = halo_rdmas()
        rx.wait_send()
        rb.wait_send()
        pl.semaphore_wait(ack_sem, 1)

    pl.semaphore_signal(exit_sem, inc=1, device_id=(my_x, 1 - my_y),
                        device_id_type=pl.DeviceIdType.MESH)
    pl.semaphore_wait(exit_sem, 1)


def kernel(x, A, B, C):
    Bb, S, D = x.shape
    N = A.shape[-1]
    dAT = jnp.exp(A).T
    Ct = jnp.transpose(C, (1, 0, 2))
    L = (jnp.eye(Bb, dtype=C.dtype)[None, :, :, None]
         * Ct[:, :, None, :]).reshape(S, Bb, Bb * N)

    Sh = S // 2
    return pl.pallas_call(
        _body,
        out_shape=jax.ShapeDtypeStruct((Bb, S, D), jnp.float32),
        in_specs=[pl.BlockSpec(memory_space=pltpu.VMEM)] * 4,
        out_specs=pl.BlockSpec(memory_space=pltpu.VMEM),
        scratch_shapes=[
            pltpu.VMEM((Bb, W, D), jnp.float32),
            pltpu.VMEM((Bb, W, N), jnp.float32),
            pltpu.VMEM((Bb, N, D), jnp.float32),
            pltpu.VMEM((Bb, Sh, D), jnp.bfloat16),
            pltpu.VMEM((Bb, Sh, D), jnp.bfloat16),
            pltpu.SemaphoreType.DMA((2,)),
            pltpu.SemaphoreType.DMA((2,)),
            pltpu.SemaphoreType.DMA((PIECES,)),
            pltpu.SemaphoreType.DMA((PIECES,)),
            pltpu.SemaphoreType.REGULAR,
            pltpu.SemaphoreType.REGULAR,
        ],
        compiler_params=pltpu.CompilerParams(collective_id=7),
    )(x, dAT, B, L)


# device time: 93705 ns/iter; 2.8851x vs baseline; 2.8851x over previous
import jax
import jax.numpy as jnp
from jax import lax
from jax.experimental import pallas as pl
from jax.experimental.pallas import tpu as pltpu

W = 32
BLK = 8
PIECES = 8


def _body(x_ref, dAT_ref, B_ref, L_ref, out_ref,
          hx_ref, hB_ref, h_ref, stg_out_ref, stg_in_ref,
          halo_send_sems, halo_recv_sems, pc_send_sems, pc_recv_sems,
          ack_sem, exit_sem):
    Bb, S, D = x_ref.shape
    N = B_ref.shape[-1]
    Sh = S // 2
    PC = Sh // PIECES
    my_x = lax.axis_index("x")
    my_y = lax.axis_index("y")
    t0 = my_y * Sh

    barrier = pltpu.get_barrier_semaphore()
    pl.semaphore_signal(barrier, inc=1, device_id=(1 - my_x, my_y),
                        device_id_type=pl.DeviceIdType.MESH)
    pl.semaphore_signal(barrier, inc=1, device_id=(my_x, 1 - my_y),
                        device_id_type=pl.DeviceIdType.MESH)
    pl.semaphore_wait(barrier, 2)

    def halo_rdmas():
        rx = pltpu.make_async_remote_copy(
            src_ref=x_ref.at[:, pl.ds(S - W, W), :],
            dst_ref=hx_ref,
            send_sem=halo_send_sems.at[0],
            recv_sem=halo_recv_sems.at[0],
            device_id=(1, my_y),
            device_id_type=pl.DeviceIdType.MESH,
        )
        rb = pltpu.make_async_remote_copy(
            src_ref=B_ref.at[:, pl.ds(S - W, W), :],
            dst_ref=hB_ref,
            send_sem=halo_send_sems.at[1],
            recv_sem=halo_recv_sems.at[1],
            device_id=(1, my_y),
            device_id_type=pl.DeviceIdType.MESH,
        )
        return rx, rb

    def piece_rdma(p):
        return pltpu.make_async_remote_copy(
            src_ref=stg_out_ref.at[:, pl.ds(p * PC, PC), :],
            dst_ref=stg_in_ref.at[:, pl.ds(p * PC, PC), :],
            send_sem=pc_send_sems.at[p],
            recv_sem=pc_recv_sems.at[p],
            device_id=(my_x, 1 - my_y),
            device_id_type=pl.DeviceIdType.MESH,
        )

    @pl.when(my_x == 0)
    def _():
        rx, rb = halo_rdmas()
        rx.start()
        rb.start()

    h_ref[...] = jnp.zeros_like(h_ref)
    dAT = dAT_ref[...][None]

    @pl.when(my_x == 1)
    def _():
        rx, rb = halo_rdmas()
        rx.wait_recv()
        rb.wait_recv()
        pl.semaphore_signal(ack_sem, inc=1, device_id=(0, my_y),
                            device_id_type=pl.DeviceIdType.MESH)

    @pl.when(my_y == 1)
    def _():
        hx_ref[...] = x_ref[:, Sh - W:Sh, :]
        hB_ref[...] = B_ref[:, Sh - W:Sh, :]

    @pl.when(jnp.logical_or(my_x == 1, my_y == 1))
    def _():
        hbt = jnp.transpose(hB_ref[...], (0, 2, 1))
        for i in range(W):
            xt = hx_ref[:, i:i + 1, :]
            bt = hbt[:, :, i:i + 1]
            h_ref[...] = h_ref[...] * dAT + xt * bt

    def step(c, carry, p):
        base = t0 + p * PC + c * BLK
        xblk = x_ref[:, pl.ds(base, BLK), :]
        bbt = jnp.transpose(B_ref[:, pl.ds(base, BLK), :], (0, 2, 1))
        lblk = L_ref[pl.ds(base, BLK)]
        ys = []
        h = h_ref[...]
        for k in range(BLK):
            xt = xblk[:, k:k + 1, :]
            bt = bbt[:, :, k:k + 1]
            h = h * dAT + xt * bt
            y = jnp.dot(lblk[k], h.reshape(Bb * N, D),
                        preferred_element_type=jnp.float32)
            ys.append(y[:, None, :])
        h_ref[...] = h
        yblk = jnp.concatenate(ys, axis=1)
        out_ref[:, pl.ds(base, BLK), :] = yblk
        stg_out_ref[:, pl.ds(p * PC + c * BLK, BLK), :] = yblk.astype(jnp.bfloat16)
        return carry

    for p in range(PIECES):
        lax.fori_loop(0, PC // BLK, lambda c, a, p=p: step(c, a, p), 0)
        piece_rdma(p).start()

    for p in range(PIECES):
        piece_rdma(p).wait_recv()
    out_ref[:, pl.ds((1 - my_y) * Sh, Sh), :] = stg_in_ref[...].astype(jnp.float32)
    for p in range(PIECES):
        piece_rdma(p).wait_send()

    @pl.when(my_x == 0)
    def _():
        rx, rb = halo_rdmas()
        rx.wait_send()
        rb.wait_send()
        pl.semaphore_wait(ack_sem, 1)

    pl.semaphore_signal(exit_sem, inc=1, device_id=(my_x, 1 - my_y),
                        device_id_type=pl.DeviceIdType.MESH)
    pl.semaphore_wait(exit_sem, 1)


def kernel(x, A, B, C):
    Bb, S, D = x.shape
    N = A.shape[-1]
    dAT = jnp.exp(A).T
    Ct = jnp.transpose(C, (1, 0, 2))
    L = (jnp.eye(Bb, dtype=C.dtype)[None, :, :, None]
         * Ct[:, :, None, :]).reshape(S, Bb, Bb * N)

    Sh = S // 2
    return pl.pallas_call(
        _body,
        out_shape=jax.ShapeDtypeStruct((Bb, S, D), jnp.float32),
        in_specs=[pl.BlockSpec(memory_space=pltpu.VMEM)] * 4,
        out_specs=pl.BlockSpec(memory_space=pltpu.VMEM),
        scratch_shapes=[
            pltpu.VMEM((Bb, W, D), jnp.float32),
            pltpu.VMEM((Bb, W, N), jnp.float32),
            pltpu.VMEM((Bb, N, D), jnp.float32),
            pltpu.VMEM((Bb, Sh, D), jnp.bfloat16),
            pltpu.VMEM((Bb, Sh, D), jnp.bfloat16),
            pltpu.SemaphoreType.DMA((2,)),
            pltpu.SemaphoreType.DMA((2,)),
            pltpu.SemaphoreType.DMA((PIECES,)),
            pltpu.SemaphoreType.DMA((PIECES,)),
            pltpu.SemaphoreType.REGULAR,
            pltpu.SemaphoreType.REGULAR,
        ],
        compiler_params=pltpu.CompilerParams(collective_id=7),
    )(x, dAT, B, L)


# device time: 82376 ns/iter; 3.2819x vs baseline; 1.1375x over previous
import jax
import jax.numpy as jnp
from jax import lax
from jax.experimental import pallas as pl
from jax.experimental.pallas import tpu as pltpu

W = 32
BLK = 32
PIECES = 8


def _body(x_ref, dAT_ref, B_ref, L_ref, out_ref,
          hx_ref, hB_ref, h_ref, stg_out_ref, stg_in_ref,
          halo_send_sems, halo_recv_sems, pc_send_sems, pc_recv_sems,
          ack_sem, exit_sem):
    Bb, S, D = x_ref.shape
    N = B_ref.shape[-1]
    Sh = S // 2
    PC = Sh // PIECES
    my_x = lax.axis_index("x")
    my_y = lax.axis_index("y")
    t0 = my_y * Sh

    barrier = pltpu.get_barrier_semaphore()
    pl.semaphore_signal(barrier, inc=1, device_id=(1 - my_x, my_y),
                        device_id_type=pl.DeviceIdType.MESH)
    pl.semaphore_signal(barrier, inc=1, device_id=(my_x, 1 - my_y),
                        device_id_type=pl.DeviceIdType.MESH)
    pl.semaphore_wait(barrier, 2)

    def halo_rdmas():
        rx = pltpu.make_async_remote_copy(
            src_ref=x_ref.at[:, pl.ds(S - W, W), :],
            dst_ref=hx_ref,
            send_sem=halo_send_sems.at[0],
            recv_sem=halo_recv_sems.at[0],
            device_id=(1, my_y),
            device_id_type=pl.DeviceIdType.MESH,
        )
        rb = pltpu.make_async_remote_copy(
            src_ref=B_ref.at[:, pl.ds(S - W, W), :],
            dst_ref=hB_ref,
            send_sem=halo_send_sems.at[1],
            recv_sem=halo_recv_sems.at[1],
            device_id=(1, my_y),
            device_id_type=pl.DeviceIdType.MESH,
        )
        return rx, rb

    def piece_rdma(p):
        return pltpu.make_async_remote_copy(
            src_ref=stg_out_ref.at[:, pl.ds(p * PC, PC), :],
            dst_ref=stg_in_ref.at[:, pl.ds(p * PC, PC), :],
            send_sem=pc_send_sems.at[p],
            recv_sem=pc_recv_sems.at[p],
            device_id=(my_x, 1 - my_y),
            device_id_type=pl.DeviceIdType.MESH,
        )

    @pl.when(my_x == 0)
    def _():
        rx, rb = halo_rdmas()
        rx.start()
        rb.start()

    h_ref[...] = jnp.zeros_like(h_ref)
    dAT = dAT_ref[...][None]

    @pl.when(my_x == 1)
    def _():
        rx, rb = halo_rdmas()
        rx.wait_recv()
        rb.wait_recv()
        pl.semaphore_signal(ack_sem, inc=1, device_id=(0, my_y),
                            device_id_type=pl.DeviceIdType.MESH)

    @pl.when(my_y == 1)
    def _():
        hx_ref[...] = x_ref[:, Sh - W:Sh, :]
        hB_ref[...] = B_ref[:, Sh - W:Sh, :]

    @pl.when(jnp.logical_or(my_x == 1, my_y == 1))
    def _():
        hbt = jnp.transpose(hB_ref[...], (0, 2, 1))
        for i in range(W):
            xt = hx_ref[:, i:i + 1, :]
            bt = hbt[:, :, i:i + 1]
            h_ref[...] = h_ref[...] * dAT + xt * bt

    def step(c, carry, p):
        base = t0 + p * PC + c * BLK
        xblk = x_ref[:, pl.ds(base, BLK), :]
        bbt = jnp.transpose(B_ref[:, pl.ds(base, BLK), :], (0, 2, 1))
        lblk = L_ref[pl.ds(base, BLK)]
        ys = []
        h = h_ref[...]
        for k in range(BLK):
            xt = xblk[:, k:k + 1, :]
            bt = bbt[:, :, k:k + 1]
            h = h * dAT + xt * bt
            y = jnp.dot(lblk[k], h.reshape(Bb * N, D),
                        preferred_element_type=jnp.float32)
            ys.append(y[:, None, :])
        h_ref[...] = h
        yblk = jnp.concatenate(ys, axis=1)
        out_ref[:, pl.ds(base, BLK), :] = yblk
        stg_out_ref[:, pl.ds(p * PC + c * BLK, BLK), :] = yblk.astype(jnp.bfloat16)
        return carry

    for p in range(PIECES):
        lax.fori_loop(0, PC // BLK, lambda c, a, p=p: step(c, a, p), 0)
        piece_rdma(p).start()

    for p in range(PIECES):
        piece_rdma(p).wait_recv()
        out_ref[:, pl.ds((1 - my_y) * Sh + p * PC, PC), :] = (
            stg_in_ref[:, pl.ds(p * PC, PC), :].astype(jnp.float32))
    for p in range(PIECES):
        piece_rdma(p).wait_send()

    @pl.when(my_x == 0)
    def _():
        rx, rb = halo_rdmas()
        rx.wait_send()
        rb.wait_send()
        pl.semaphore_wait(ack_sem, 1)

    pl.semaphore_signal(exit_sem, inc=1, device_id=(my_x, 1 - my_y),
                        device_id_type=pl.DeviceIdType.MESH)
    pl.semaphore_wait(exit_sem, 1)


def kernel(x, A, B, C):
    Bb, S, D = x.shape
    N = A.shape[-1]
    dAT = jnp.exp(A).T
    Ct = jnp.transpose(C, (1, 0, 2))
    L = (jnp.eye(Bb, dtype=C.dtype)[None, :, :, None]
         * Ct[:, :, None, :]).reshape(S, Bb, Bb * N)

    Sh = S // 2
    return pl.pallas_call(
        _body,
        out_shape=jax.ShapeDtypeStruct((Bb, S, D), jnp.float32),
        in_specs=[pl.BlockSpec(memory_space=pltpu.VMEM)] * 4,
        out_specs=pl.BlockSpec(memory_space=pltpu.VMEM),
        scratch_shapes=[
            pltpu.VMEM((Bb, W, D), jnp.float32),
            pltpu.VMEM((Bb, W, N), jnp.float32),
            pltpu.VMEM((Bb, N, D), jnp.float32),
            pltpu.VMEM((Bb, Sh, D), jnp.bfloat16),
            pltpu.VMEM((Bb, Sh, D), jnp.bfloat16),
            pltpu.SemaphoreType.DMA((2,)),
            pltpu.SemaphoreType.DMA((2,)),
            pltpu.SemaphoreType.DMA((PIECES,)),
            pltpu.SemaphoreType.DMA((PIECES,)),
            pltpu.SemaphoreType.REGULAR,
            pltpu.SemaphoreType.REGULAR,
        ],
        compiler_params=pltpu.CompilerParams(collective_id=7),
    )(x, dAT, B, L)


# device time: 72509 ns/iter; 3.7284x vs baseline; 1.1361x over previous
import jax
import jax.numpy as jnp
from jax import lax
from jax.experimental import pallas as pl
from jax.experimental.pallas import tpu as pltpu

W = 16
BLK = 32
PIECES = 16


def _body(x_ref, dAT_ref, B_ref, L_ref, out_ref,
          hx_ref, hB_ref, h_ref, stg_out_ref, stg_in_ref,
          halo_send_sems, halo_recv_sems, pc_send_sems, pc_recv_sems,
          ack_sem, exit_sem):
    Bb, S, D = x_ref.shape
    N = B_ref.shape[-1]
    Sh = S // 2
    PC = Sh // PIECES
    my_x = lax.axis_index("x")
    my_y = lax.axis_index("y")
    t0 = my_y * Sh

    barrier = pltpu.get_barrier_semaphore()
    pl.semaphore_signal(barrier, inc=1, device_id=(1 - my_x, my_y),
                        device_id_type=pl.DeviceIdType.MESH)
    pl.semaphore_signal(barrier, inc=1, device_id=(my_x, 1 - my_y),
                        device_id_type=pl.DeviceIdType.MESH)
    pl.semaphore_wait(barrier, 2)

    def halo_rdmas():
        rx = pltpu.make_async_remote_copy(
            src_ref=x_ref.at[:, pl.ds(S - W, W), :],
            dst_ref=hx_ref,
            send_sem=halo_send_sems.at[0],
            recv_sem=halo_recv_sems.at[0],
            device_id=(1, my_y),
            device_id_type=pl.DeviceIdType.MESH,
        )
        rb = pltpu.make_async_remote_copy(
            src_ref=B_ref.at[:, pl.ds(S - W, W), :],
            dst_ref=hB_ref,
            send_sem=halo_send_sems.at[1],
            recv_sem=halo_recv_sems.at[1],
            device_id=(1, my_y),
            device_id_type=pl.DeviceIdType.MESH,
        )
        return rx, rb

    def piece_rdma(p):
        return pltpu.make_async_remote_copy(
            src_ref=stg_out_ref.at[:, pl.ds(p * PC, PC), :],
            dst_ref=stg_in_ref.at[:, pl.ds(p * PC, PC), :],
            send_sem=pc_send_sems.at[p],
            recv_sem=pc_recv_sems.at[p],
            device_id=(my_x, 1 - my_y),
            device_id_type=pl.DeviceIdType.MESH,
        )

    @pl.when(my_x == 0)
    def _():
        rx, rb = halo_rdmas()
        rx.start()
        rb.start()

    h_ref[...] = jnp.zeros_like(h_ref)
    dAT = dAT_ref[...][None]

    @pl.when(my_x == 1)
    def _():
        rx, rb = halo_rdmas()
        rx.wait_recv()
        rb.wait_recv()
        pl.semaphore_signal(ack_sem, inc=1, device_id=(0, my_y),
                            device_id_type=pl.DeviceIdType.MESH)

    @pl.when(my_y == 1)
    def _():
        hx_ref[...] = x_ref[:, Sh - W:Sh, :]
        hB_ref[...] = B_ref[:, Sh - W:Sh, :]

    @pl.when(jnp.logical_or(my_x == 1, my_y == 1))
    def _():
        hxb = hx_ref[...].astype(jnp.bfloat16)
        hbt = jnp.transpose(hB_ref[...], (0, 2, 1)).astype(jnp.bfloat16)
        for i in range(W):
            xt = hxb[:, i:i + 1, :]
            bt = hbt[:, :, i:i + 1]
            h_ref[...] = h_ref[...] * dAT + xt * bt

    def step(c, carry, p):
        base = t0 + p * PC + c * BLK
        xblk = x_ref[:, pl.ds(base, BLK), :].astype(jnp.bfloat16)
        bbt = jnp.transpose(B_ref[:, pl.ds(base, BLK), :],
                            (0, 2, 1)).astype(jnp.bfloat16)
        lblk = L_ref[pl.ds(base, BLK)]
        ys = []
        h = h_ref[...]
        for k in range(BLK):
            xt = xblk[:, k:k + 1, :]
            bt = bbt[:, :, k:k + 1]
            h = h * dAT + xt * bt
            y = jnp.dot(lblk[k], h.reshape(Bb * N, D),
                        preferred_element_type=jnp.float32)
            ys.append(y[:, None, :])
        h_ref[...] = h
        yblk = jnp.concatenate(ys, axis=1)
        out_ref[:, pl.ds(base, BLK), :] = yblk
        stg_out_ref[:, pl.ds(p * PC + c * BLK, BLK), :] = yblk.astype(jnp.bfloat16)
        return carry

    for p in range(PIECES):
        lax.fori_loop(0, PC // BLK, lambda c, a, p=p: step(c, a, p), 0)
        piece_rdma(p).start()

    for p in range(PIECES):
        piece_rdma(p).wait_recv()
        out_ref[:, pl.ds((1 - my_y) * Sh + p * PC, PC), :] = (
            stg_in_ref[:, pl.ds(p * PC, PC), :].astype(jnp.float32))
    for p in range(PIECES):
        piece_rdma(p).wait_send()

    @pl.when(my_x == 0)
    def _():
        rx, rb = halo_rdmas()
        rx.wait_send()
        rb.wait_send()
        pl.semaphore_wait(ack_sem, 1)

    pl.semaphore_signal(exit_sem, inc=1, device_id=(my_x, 1 - my_y),
                        device_id_type=pl.DeviceIdType.MESH)
    pl.semaphore_wait(exit_sem, 1)


def kernel(x, A, B, C):
    Bb, S, D = x.shape
    N = A.shape[-1]
    dAT = jnp.exp(A).T.astype(jnp.bfloat16)
    Ct = jnp.transpose(C, (1, 0, 2))
    L = (jnp.eye(Bb, dtype=C.dtype)[None, :, :, None]
         * Ct[:, :, None, :]).reshape(S, Bb, Bb * N).astype(jnp.bfloat16)

    Sh = S // 2
    return pl.pallas_call(
        _body,
        out_shape=jax.ShapeDtypeStruct((Bb, S, D), jnp.float32),
        in_specs=[pl.BlockSpec(memory_space=pltpu.VMEM)] * 4,
        out_specs=pl.BlockSpec(memory_space=pltpu.VMEM),
        scratch_shapes=[
            pltpu.VMEM((Bb, W, D), jnp.float32),
            pltpu.VMEM((Bb, W, N), jnp.float32),
            pltpu.VMEM((Bb, N, D), jnp.bfloat16),
            pltpu.VMEM((Bb, Sh, D), jnp.bfloat16),
            pltpu.VMEM((Bb, Sh, D), jnp.bfloat16),
            pltpu.SemaphoreType.DMA((2,)),
            pltpu.SemaphoreType.DMA((2,)),
            pltpu.SemaphoreType.DMA((PIECES,)),
            pltpu.SemaphoreType.DMA((PIECES,)),
            pltpu.SemaphoreType.REGULAR,
            pltpu.SemaphoreType.REGULAR,
        ],
        compiler_params=pltpu.CompilerParams(collective_id=7),
    )(x, dAT, B, L)
